# Initial kernel scaffold; baseline (speedup 1.0000x reference)
import functools

import jax
import jax.numpy as jnp
from jax import lax
from jax.experimental import pallas as pl
from jax.experimental.pallas import tpu as pltpu
from jax.experimental.pallas import tpu_sc as plsc

N = 10000
E = 320000
D = 128

NC = 2
NS = 16
CH = 80

EPT = E // (NC * NS)
JPT = EPT // CH
NPT = N // NS

_mesh = plsc.VectorSubcoreMesh(core_axis_name="c", subcore_axis_name="s")


@functools.partial(
    pl.kernel,
    mesh=_mesh,
    out_type=(
        jax.ShapeDtypeStruct((N, D), jnp.float32),
        jax.ShapeDtypeStruct((N, D), jnp.float32),
    ),
    scratch_types=[
        pltpu.VMEM((JPT, CH), jnp.int32),
        pltpu.VMEM((JPT, CH), jnp.int32),
        pltpu.VMEM((CH, D), jnp.float32),
        pltpu.VMEM_SHARED((N, D), jnp.float32),
        pltpu.SemaphoreType.DMA,
    ],
)
def _sc_segment_sum(h_hbm, src_hbm, dst_hbm, zero_hbm,
                    out0_hbm, out1_hbm, srcv, dstv, rows, acc, sem):
    c = lax.axis_index("c")
    s = lax.axis_index("s")
    wid = c * NS + s

    pltpu.sync_copy(zero_hbm.at[pl.ds(s * NPT, NPT)],
                    acc.at[pl.ds(s * NPT, NPT)])
    row0 = wid * JPT
    pltpu.sync_copy(src_hbm.at[pl.ds(row0, JPT)], srcv)
    pltpu.sync_copy(dst_hbm.at[pl.ds(row0, JPT)], dstv)
    plsc.subcore_barrier()

    def body(j, carry):
        pltpu.async_copy(h_hbm.at[srcv.at[j]], rows, sem).wait()
        pltpu.sync_copy(rows, acc.at[dstv.at[j]], add=True)
        return carry

    lax.fori_loop(0, JPT, body, 0)
    plsc.subcore_barrier()

    @pl.when(c == 0)
    def _():
        pltpu.sync_copy(acc.at[pl.ds(s * NPT, NPT)],
                        out0_hbm.at[pl.ds(s * NPT, NPT)])

    @pl.when(c == 1)
    def _():
        pltpu.sync_copy(acc.at[pl.ds(s * NPT, NPT)],
                        out1_hbm.at[pl.ds(s * NPT, NPT)])


BN = 1000


def _mlp_body(p0, p1, w1, b1, w2, o):
    a = p0[...] + p1[...]
    t = jnp.dot(a, w1[...], preferred_element_type=jnp.float32) + b1[...]
    t = jnp.maximum(t, 0.0)
    o[...] = jnp.dot(t, w2[...], preferred_element_type=jnp.float32)


_tc_mlp = pl.pallas_call(
    _mlp_body,
    grid=(N // BN,),
    in_specs=[
        pl.BlockSpec((BN, D), lambda i: (i, 0)),
        pl.BlockSpec((BN, D), lambda i: (i, 0)),
        pl.BlockSpec((D, D), lambda i: (0, 0)),
        pl.BlockSpec((1, D), lambda i: (0, 0)),
        pl.BlockSpec((D, D), lambda i: (0, 0)),
    ],
    out_specs=pl.BlockSpec((BN, D), lambda i: (i, 0)),
    out_shape=jax.ShapeDtypeStruct((N, D), jnp.float32),
)


def _bias_body(p0, p1, b2, o):
    o[...] = p0[...] + p1[...] + b2[...]


_tc_bias = pl.pallas_call(
    _bias_body,
    grid=(N // BN,),
    in_specs=[
        pl.BlockSpec((BN, D), lambda i: (i, 0)),
        pl.BlockSpec((BN, D), lambda i: (i, 0)),
        pl.BlockSpec((1, D), lambda i: (0, 0)),
    ],
    out_specs=pl.BlockSpec((BN, D), lambda i: (i, 0)),
    out_shape=jax.ShapeDtypeStruct((N, D), jnp.float32),
)


def kernel(x, edge_index, W1, b1, W2, b2):
    src = edge_index[0].reshape(E // CH, CH)
    dst = edge_index[1].reshape(E // CH, CH)
    zeros = jnp.zeros((N, D), jnp.float32)
    p0, p1 = _sc_segment_sum(x, src, dst, zeros)
    h2 = _tc_mlp(p0, p1, W1, b1.reshape(1, D), W2)
    q0, q1 = _sc_segment_sum(h2, src, dst, zeros)
    return _tc_bias(q0, q1, b2.reshape(1, D))


# speedup vs baseline: 7.3130x; 7.3130x over previous
import functools

import jax
import jax.numpy as jnp
from jax import lax
from jax.experimental import pallas as pl
from jax.experimental.pallas import tpu as pltpu
from jax.experimental.pallas import tpu_sc as plsc

N = 10000
E = 320000
D = 128

NC = 2
NS = 16
CH = 80

EPT = E // (NC * NS)
JPT = EPT // CH
S0 = (N // NS) // 8 * 8
NTAIL = N - NS * S0

_mesh = plsc.VectorSubcoreMesh(core_axis_name="c", subcore_axis_name="s")


@functools.partial(
    pl.kernel,
    mesh=_mesh,
    out_type=(
        jax.ShapeDtypeStruct((N, D), jnp.float32),
        jax.ShapeDtypeStruct((N, D), jnp.float32),
    ),
    scratch_types=[
        pltpu.VMEM((JPT, CH), jnp.int32),
        pltpu.VMEM((JPT, CH), jnp.int32),
        pltpu.VMEM((CH, D), jnp.float32),
        pltpu.VMEM_SHARED((N, D), jnp.float32),
        pltpu.SemaphoreType.DMA,
    ],
)
def _sc_segment_sum(h_hbm, src_hbm, dst_hbm, zero_hbm,
                    out0_hbm, out1_hbm, srcv, dstv, rows, acc, sem):
    c = lax.axis_index("c")
    s = lax.axis_index("s")
    wid = c * NS + s

    pltpu.sync_copy(zero_hbm.at[pl.ds(s * S0, S0)], acc.at[pl.ds(s * S0, S0)])

    @pl.when(s == NS - 1)
    def _():
        pltpu.sync_copy(zero_hbm.at[pl.ds(NS * S0, NTAIL)],
                        acc.at[pl.ds(NS * S0, NTAIL)])

    pltpu.sync_copy(src_hbm.at[wid], srcv)
    pltpu.sync_copy(dst_hbm.at[wid], dstv)
    plsc.subcore_barrier()

    def body(j, carry):
        pltpu.async_copy(h_hbm.at[srcv.at[j]], rows, sem).wait()
        pltpu.sync_copy(rows, acc.at[dstv.at[j]], add=True)
        return carry

    lax.fori_loop(0, JPT, body, 0)
    plsc.subcore_barrier()

    @pl.when(c == 0)
    def _():
        pltpu.sync_copy(acc.at[pl.ds(s * S0, S0)],
                        out0_hbm.at[pl.ds(s * S0, S0)])

        @pl.when(s == NS - 1)
        def _():
            pltpu.sync_copy(acc.at[pl.ds(NS * S0, NTAIL)],
                            out0_hbm.at[pl.ds(NS * S0, NTAIL)])

    @pl.when(c == 1)
    def _():
        pltpu.sync_copy(acc.at[pl.ds(s * S0, S0)],
                        out1_hbm.at[pl.ds(s * S0, S0)])

        @pl.when(s == NS - 1)
        def _():
            pltpu.sync_copy(acc.at[pl.ds(NS * S0, NTAIL)],
                            out1_hbm.at[pl.ds(NS * S0, NTAIL)])


BN = 1000


def _mlp_body(p0, p1, w1, b1, w2, o):
    a = p0[...] + p1[...]
    t = jnp.dot(a, w1[...], preferred_element_type=jnp.float32) + b1[...]
    t = jnp.maximum(t, 0.0)
    o[...] = jnp.dot(t, w2[...], preferred_element_type=jnp.float32)


_tc_mlp = pl.pallas_call(
    _mlp_body,
    grid=(N // BN,),
    in_specs=[
        pl.BlockSpec((BN, D), lambda i: (i, 0)),
        pl.BlockSpec((BN, D), lambda i: (i, 0)),
        pl.BlockSpec((D, D), lambda i: (0, 0)),
        pl.BlockSpec((1, D), lambda i: (0, 0)),
        pl.BlockSpec((D, D), lambda i: (0, 0)),
    ],
    out_specs=pl.BlockSpec((BN, D), lambda i: (i, 0)),
    out_shape=jax.ShapeDtypeStruct((N, D), jnp.float32),
)


def _bias_body(p0, p1, b2, o):
    o[...] = p0[...] + p1[...] + b2[...]


_tc_bias = pl.pallas_call(
    _bias_body,
    grid=(N // BN,),
    in_specs=[
        pl.BlockSpec((BN, D), lambda i: (i, 0)),
        pl.BlockSpec((BN, D), lambda i: (i, 0)),
        pl.BlockSpec((1, D), lambda i: (0, 0)),
    ],
    out_specs=pl.BlockSpec((BN, D), lambda i: (i, 0)),
    out_shape=jax.ShapeDtypeStruct((N, D), jnp.float32),
)


def kernel(x, edge_index, W1, b1, W2, b2):
    src = edge_index[0].reshape(NC * NS, JPT, CH)
    dst = edge_index[1].reshape(NC * NS, JPT, CH)
    zeros = jnp.zeros((N, D), jnp.float32)
    p0, p1 = _sc_segment_sum(x, src, dst, zeros)
    h2 = _tc_mlp(p0, p1, W1, b1.reshape(1, D), W2)
    q0, q1 = _sc_segment_sum(h2, src, dst, zeros)
    return _tc_bias(q0, q1, b2.reshape(1, D))


# speedup vs baseline: 10.3263x; 1.4121x over previous
import functools

import jax
import jax.numpy as jnp
from jax import lax
from jax.experimental import pallas as pl
from jax.experimental.pallas import tpu as pltpu
from jax.experimental.pallas import tpu_sc as plsc

N = 10000
E = 320000
D = 128

NC = 2
NS = 16
CH = 125
W = 16

EPT = E // (NC * NS)
JPT = EPT // CH
S0 = (N // NS) // 8 * 8
NTAIL = N - NS * S0

_mesh = plsc.VectorSubcoreMesh(core_axis_name="c", subcore_axis_name="s")


@functools.partial(
    pl.kernel,
    mesh=_mesh,
    out_type=(
        jax.ShapeDtypeStruct((N, D), jnp.float32),
        jax.ShapeDtypeStruct((N, D), jnp.float32),
    ),
    scratch_types=[
        pltpu.VMEM((W, CH), jnp.int32),
        pltpu.VMEM((W, CH), jnp.int32),
        pltpu.VMEM((CH, D), jnp.float32),
        pltpu.VMEM((CH, D), jnp.float32),
        pltpu.VMEM_SHARED((N, D), jnp.float32),
        pltpu.SemaphoreType.DMA,
        pltpu.SemaphoreType.DMA,
    ],
)
def _sc_segment_sum(h_hbm, src_hbm, dst_hbm, zero_hbm,
                    out0_hbm, out1_hbm, srcv, dstv, rows0, rows1, acc, g0, g1):
    c = lax.axis_index("c")
    s = lax.axis_index("s")
    wid = c * NS + s

    pltpu.sync_copy(zero_hbm.at[pl.ds(s * S0, S0)], acc.at[pl.ds(s * S0, S0)])

    @pl.when(s == NS - 1)
    def _():
        pltpu.sync_copy(zero_hbm.at[pl.ds(NS * S0, NTAIL)],
                        acc.at[pl.ds(NS * S0, NTAIL)])

    plsc.subcore_barrier()

    def wbody(w, carry):
        pltpu.sync_copy(src_hbm.at[wid, pl.ds(w * W, W)], srcv)
        pltpu.sync_copy(dst_hbm.at[wid, pl.ds(w * W, W)], dstv)
        pltpu.async_copy(h_hbm.at[srcv.at[0]], rows0, g0)

        def body(i, carry):
            j = i * 2
            pltpu.make_async_copy(h_hbm.at[srcv.at[j]], rows0, g0).wait()
            pltpu.async_copy(h_hbm.at[srcv.at[j + 1]], rows1, g1)
            pltpu.sync_copy(rows0, acc.at[dstv.at[j]], add=True)

            pltpu.make_async_copy(h_hbm.at[srcv.at[j + 1]], rows1, g1).wait()

            @pl.when(j + 2 < W)
            def _():
                pltpu.async_copy(h_hbm.at[srcv.at[j + 2]], rows0, g0)

            pltpu.sync_copy(rows1, acc.at[dstv.at[j + 1]], add=True)
            return carry

        lax.fori_loop(0, W // 2, body, 0)
        return carry

    lax.fori_loop(0, JPT // W, wbody, 0)
    plsc.subcore_barrier()

    @pl.when(c == 0)
    def _():
        pltpu.sync_copy(acc.at[pl.ds(s * S0, S0)],
                        out0_hbm.at[pl.ds(s * S0, S0)])

        @pl.when(s == NS - 1)
        def _():
            pltpu.sync_copy(acc.at[pl.ds(NS * S0, NTAIL)],
                            out0_hbm.at[pl.ds(NS * S0, NTAIL)])

    @pl.when(c == 1)
    def _():
        pltpu.sync_copy(acc.at[pl.ds(s * S0, S0)],
                        out1_hbm.at[pl.ds(s * S0, S0)])

        @pl.when(s == NS - 1)
        def _():
            pltpu.sync_copy(acc.at[pl.ds(NS * S0, NTAIL)],
                            out1_hbm.at[pl.ds(NS * S0, NTAIL)])


BN = 1000


def _mlp_body(p0, p1, w1, b1, w2, o):
    a = p0[...] + p1[...]
    t = jnp.dot(a, w1[...], preferred_element_type=jnp.float32) + b1[...]
    t = jnp.maximum(t, 0.0)
    o[...] = jnp.dot(t, w2[...], preferred_element_type=jnp.float32)


_tc_mlp = pl.pallas_call(
    _mlp_body,
    grid=(N // BN,),
    in_specs=[
        pl.BlockSpec((BN, D), lambda i: (i, 0)),
        pl.BlockSpec((BN, D), lambda i: (i, 0)),
        pl.BlockSpec((D, D), lambda i: (0, 0)),
        pl.BlockSpec((1, D), lambda i: (0, 0)),
        pl.BlockSpec((D, D), lambda i: (0, 0)),
    ],
    out_specs=pl.BlockSpec((BN, D), lambda i: (i, 0)),
    out_shape=jax.ShapeDtypeStruct((N, D), jnp.float32),
)


def _bias_body(p0, p1, b2, o):
    o[...] = p0[...] + p1[...] + b2[...]


_tc_bias = pl.pallas_call(
    _bias_body,
    grid=(N // BN,),
    in_specs=[
        pl.BlockSpec((BN, D), lambda i: (i, 0)),
        pl.BlockSpec((BN, D), lambda i: (i, 0)),
        pl.BlockSpec((1, D), lambda i: (0, 0)),
    ],
    out_specs=pl.BlockSpec((BN, D), lambda i: (i, 0)),
    out_shape=jax.ShapeDtypeStruct((N, D), jnp.float32),
)


def kernel(x, edge_index, W1, b1, W2, b2):
    src = edge_index[0].reshape(NC * NS, JPT, CH)
    dst = edge_index[1].reshape(NC * NS, JPT, CH)
    zeros = jnp.zeros((N, D), jnp.float32)
    p0, p1 = _sc_segment_sum(x, src, dst, zeros)
    h2 = _tc_mlp(p0, p1, W1, b1.reshape(1, D), W2)
    q0, q1 = _sc_segment_sum(h2, src, dst, zeros)
    return _tc_bias(q0, q1, b2.reshape(1, D))


# speedup vs baseline: 10.3497x; 1.0023x over previous
import functools

import jax
import jax.numpy as jnp
from jax import lax
from jax.experimental import pallas as pl
from jax.experimental.pallas import tpu as pltpu
from jax.experimental.pallas import tpu_sc as plsc

N = 10000
E = 320000
D = 128

NC = 2
NS = 16
CH = 125
W = 16

EPT = E // (NC * NS)
JPT = EPT // CH
S0 = (N // NS) // 8 * 8
NTAIL = N - NS * S0

_mesh = plsc.VectorSubcoreMesh(core_axis_name="c", subcore_axis_name="s")


@functools.partial(
    pl.kernel,
    mesh=_mesh,
    out_type=(
        jax.ShapeDtypeStruct((N, D), jnp.float32),
        jax.ShapeDtypeStruct((N, D), jnp.float32),
    ),
    scratch_types=[
        pltpu.VMEM((W, CH), jnp.int32),
        pltpu.VMEM((W, CH), jnp.int32),
        pltpu.VMEM((CH, D), jnp.float32),
        pltpu.VMEM((CH, D), jnp.float32),
        pltpu.VMEM_SHARED((N, D), jnp.float32),
        pltpu.SemaphoreType.DMA,
        pltpu.SemaphoreType.DMA,
        pltpu.SemaphoreType.DMA,
        pltpu.SemaphoreType.DMA,
    ],
)
def _sc_segment_sum(h_hbm, src_hbm, dst_hbm, zero_hbm, out0_hbm, out1_hbm,
                    srcv, dstv, rows0, rows1, acc, g0, g1, s0, s1):
    c = lax.axis_index("c")
    s = lax.axis_index("s")
    wid = c * NS + s

    pltpu.sync_copy(zero_hbm.at[pl.ds(s * S0, S0)], acc.at[pl.ds(s * S0, S0)])

    @pl.when(s == NS - 1)
    def _():
        pltpu.sync_copy(zero_hbm.at[pl.ds(NS * S0, NTAIL)],
                        acc.at[pl.ds(NS * S0, NTAIL)])

    plsc.subcore_barrier()

    def wbody(w, carry):
        pltpu.sync_copy(src_hbm.at[wid, pl.ds(w * W, W)], srcv)
        pltpu.sync_copy(dst_hbm.at[wid, pl.ds(w * W, W)], dstv)
        pltpu.async_copy(h_hbm.at[srcv.at[0]], rows0, g0)

        @pl.when(w == 0)
        def _():
            pltpu.sync_copy(zero_hbm.at[srcv.at[0]], rows1)
            pltpu.async_copy(rows1, acc.at[dstv.at[0]], s1, add=True)

        def body(i, carry):
            j = i * 2
            pltpu.make_async_copy(h_hbm.at[srcv.at[j]], rows0, g0).wait()
            pltpu.async_copy(rows0, acc.at[dstv.at[j]], s0, add=True)
            pltpu.make_async_copy(rows1, acc.at[dstv.at[j]], s1).wait()
            pltpu.async_copy(h_hbm.at[srcv.at[j + 1]], rows1, g1)

            pltpu.make_async_copy(h_hbm.at[srcv.at[j + 1]], rows1, g1).wait()
            pltpu.async_copy(rows1, acc.at[dstv.at[j + 1]], s1, add=True)
            pltpu.make_async_copy(rows0, acc.at[dstv.at[j]], s0).wait()

            @pl.when(j + 2 < W)
            def _():
                pltpu.async_copy(h_hbm.at[srcv.at[j + 2]], rows0, g0)

            return carry

        lax.fori_loop(0, W // 2, body, 0)
        return carry

    lax.fori_loop(0, JPT // W, wbody, 0)
    pltpu.make_async_copy(rows1, acc.at[dstv.at[0]], s1).wait()
    plsc.subcore_barrier()

    @pl.when(c == 0)
    def _():
        pltpu.sync_copy(acc.at[pl.ds(s * S0, S0)],
                        out0_hbm.at[pl.ds(s * S0, S0)])

        @pl.when(s == NS - 1)
        def _():
            pltpu.sync_copy(acc.at[pl.ds(NS * S0, NTAIL)],
                            out0_hbm.at[pl.ds(NS * S0, NTAIL)])

    @pl.when(c == 1)
    def _():
        pltpu.sync_copy(acc.at[pl.ds(s * S0, S0)],
                        out1_hbm.at[pl.ds(s * S0, S0)])

        @pl.when(s == NS - 1)
        def _():
            pltpu.sync_copy(acc.at[pl.ds(NS * S0, NTAIL)],
                            out1_hbm.at[pl.ds(NS * S0, NTAIL)])


BN = 1000


def _mlp_body(p0, p1, w1, b1, w2, o):
    a = p0[...] + p1[...]
    t = jnp.dot(a, w1[...], preferred_element_type=jnp.float32) + b1[...]
    t = jnp.maximum(t, 0.0)
    o[...] = jnp.dot(t, w2[...], preferred_element_type=jnp.float32)


_tc_mlp = pl.pallas_call(
    _mlp_body,
    grid=(N // BN,),
    in_specs=[
        pl.BlockSpec((BN, D), lambda i: (i, 0)),
        pl.BlockSpec((BN, D), lambda i: (i, 0)),
        pl.BlockSpec((D, D), lambda i: (0, 0)),
        pl.BlockSpec((1, D), lambda i: (0, 0)),
        pl.BlockSpec((D, D), lambda i: (0, 0)),
    ],
    out_specs=pl.BlockSpec((BN, D), lambda i: (i, 0)),
    out_shape=jax.ShapeDtypeStruct((N, D), jnp.float32),
)


def _bias_body(p0, p1, b2, o):
    o[...] = p0[...] + p1[...] + b2[...]


_tc_bias = pl.pallas_call(
    _bias_body,
    grid=(N // BN,),
    in_specs=[
        pl.BlockSpec((BN, D), lambda i: (i, 0)),
        pl.BlockSpec((BN, D), lambda i: (i, 0)),
        pl.BlockSpec((1, D), lambda i: (0, 0)),
    ],
    out_specs=pl.BlockSpec((BN, D), lambda i: (i, 0)),
    out_shape=jax.ShapeDtypeStruct((N, D), jnp.float32),
)


def kernel(x, edge_index, W1, b1, W2, b2):
    src = edge_index[0].reshape(NC * NS, JPT, CH)
    dst = edge_index[1].reshape(NC * NS, JPT, CH)
    zeros = jnp.zeros((N, D), jnp.float32)
    p0, p1 = _sc_segment_sum(x, src, dst, zeros)
    h2 = _tc_mlp(p0, p1, W1, b1.reshape(1, D), W2)
    q0, q1 = _sc_segment_sum(h2, src, dst, zeros)
    return _tc_bias(q0, q1, b2.reshape(1, D))


# speedup vs baseline: 11.0923x; 1.0718x over previous
import functools

import jax
import jax.numpy as jnp
from jax import lax
from jax.experimental import pallas as pl
from jax.experimental.pallas import tpu as pltpu
from jax.experimental.pallas import tpu_sc as plsc

N = 10000
E = 320000
D = 128

NC = 2
NS = 16
CH = 125
W = 40

EPT = E // (NC * NS)
JPT = EPT // CH
S0 = (N // NS) // 8 * 8
NTAIL = N - NS * S0

_mesh = plsc.VectorSubcoreMesh(core_axis_name="c", subcore_axis_name="s")


@functools.partial(
    pl.kernel,
    mesh=_mesh,
    out_type=(
        jax.ShapeDtypeStruct((N, D), jnp.float32),
        jax.ShapeDtypeStruct((N, D), jnp.float32),
    ),
    scratch_types=[
        pltpu.VMEM((W, CH), jnp.int32),
        pltpu.VMEM((W, CH), jnp.int32),
        pltpu.VMEM((CH, D), jnp.float32),
        pltpu.VMEM((CH, D), jnp.float32),
        pltpu.VMEM_SHARED((N, D), jnp.float32),
        pltpu.SemaphoreType.DMA,
        pltpu.SemaphoreType.DMA,
        pltpu.SemaphoreType.DMA,
        pltpu.SemaphoreType.DMA,
    ],
)
def _sc_segment_sum(h_hbm, ei_hbm, zero_hbm, out0_hbm, out1_hbm,
                    srcv, dstv, rows0, rows1, acc, g0, g1, s0, s1):
    c = lax.axis_index("c")
    s = lax.axis_index("s")
    wid = c * NS + s

    pltpu.async_copy(ei_hbm.at[0, wid, pl.ds(0, W)], srcv, g0)
    pltpu.async_copy(ei_hbm.at[1, wid, pl.ds(0, W)], dstv, g1)

    pltpu.sync_copy(zero_hbm.at[pl.ds(s * S0, S0)], acc.at[pl.ds(s * S0, S0)])

    @pl.when(s == NS - 1)
    def _():
        pltpu.sync_copy(zero_hbm.at[pl.ds(NS * S0, NTAIL)],
                        acc.at[pl.ds(NS * S0, NTAIL)])

    pltpu.make_async_copy(ei_hbm.at[0, wid, pl.ds(0, W)], srcv, g0).wait()
    pltpu.make_async_copy(ei_hbm.at[1, wid, pl.ds(0, W)], dstv, g1).wait()
    plsc.subcore_barrier()

    def wbody(w, carry):
        @pl.when(w > 0)
        def _():
            pltpu.sync_copy(ei_hbm.at[0, wid, pl.ds(w * W, W)], srcv)
            pltpu.sync_copy(ei_hbm.at[1, wid, pl.ds(w * W, W)], dstv)

        pltpu.async_copy(h_hbm.at[srcv.at[0]], rows0, g0)

        @pl.when(w == 0)
        def _():
            pltpu.sync_copy(zero_hbm.at[dstv.at[0]], rows1)
            pltpu.async_copy(rows1, acc.at[dstv.at[0]], s1, add=True)

        def body(i, carry):
            j = i * 2
            pltpu.make_async_copy(h_hbm.at[srcv.at[j]], rows0, g0).wait()
            pltpu.async_copy(rows0, acc.at[dstv.at[j]], s0, add=True)
            pltpu.make_async_copy(rows1, acc.at[dstv.at[j]], s1).wait()
            pltpu.async_copy(h_hbm.at[srcv.at[j + 1]], rows1, g1)

            pltpu.make_async_copy(h_hbm.at[srcv.at[j + 1]], rows1, g1).wait()
            pltpu.async_copy(rows1, acc.at[dstv.at[j + 1]], s1, add=True)
            pltpu.make_async_copy(rows0, acc.at[dstv.at[j]], s0).wait()

            @pl.when(j + 2 < W)
            def _():
                pltpu.async_copy(h_hbm.at[srcv.at[j + 2]], rows0, g0)

            return carry

        lax.fori_loop(0, W // 2, body, 0)
        return carry

    lax.fori_loop(0, JPT // W, wbody, 0)
    pltpu.make_async_copy(rows1, acc.at[dstv.at[0]], s1).wait()
    plsc.subcore_barrier()

    @pl.when(c == 0)
    def _():
        pltpu.sync_copy(acc.at[pl.ds(s * S0, S0)],
                        out0_hbm.at[pl.ds(s * S0, S0)])

        @pl.when(s == NS - 1)
        def _():
            pltpu.sync_copy(acc.at[pl.ds(NS * S0, NTAIL)],
                            out0_hbm.at[pl.ds(NS * S0, NTAIL)])

    @pl.when(c == 1)
    def _():
        pltpu.sync_copy(acc.at[pl.ds(s * S0, S0)],
                        out1_hbm.at[pl.ds(s * S0, S0)])

        @pl.when(s == NS - 1)
        def _():
            pltpu.sync_copy(acc.at[pl.ds(NS * S0, NTAIL)],
                            out1_hbm.at[pl.ds(NS * S0, NTAIL)])


BN = 1000


def _mlp_body(p0, p1, w1, b1, w2, o):
    a = p0[...] + p1[...]
    t = jnp.dot(a, w1[...], preferred_element_type=jnp.float32) + b1[...]
    t = jnp.maximum(t, 0.0)
    o[...] = jnp.dot(t, w2[...], preferred_element_type=jnp.float32)


_tc_mlp = pl.pallas_call(
    _mlp_body,
    grid=(N // BN,),
    in_specs=[
        pl.BlockSpec((BN, D), lambda i: (i, 0)),
        pl.BlockSpec((BN, D), lambda i: (i, 0)),
        pl.BlockSpec((D, D), lambda i: (0, 0)),
        pl.BlockSpec((1, D), lambda i: (0, 0)),
        pl.BlockSpec((D, D), lambda i: (0, 0)),
    ],
    out_specs=pl.BlockSpec((BN, D), lambda i: (i, 0)),
    out_shape=jax.ShapeDtypeStruct((N, D), jnp.float32),
)


def _bias_body(p0, p1, b2, o):
    o[...] = p0[...] + p1[...] + b2[...]


_tc_bias = pl.pallas_call(
    _bias_body,
    grid=(N // BN,),
    in_specs=[
        pl.BlockSpec((BN, D), lambda i: (i, 0)),
        pl.BlockSpec((BN, D), lambda i: (i, 0)),
        pl.BlockSpec((1, D), lambda i: (0, 0)),
    ],
    out_specs=pl.BlockSpec((BN, D), lambda i: (i, 0)),
    out_shape=jax.ShapeDtypeStruct((N, D), jnp.float32),
)


def kernel(x, edge_index, W1, b1, W2, b2):
    ei = edge_index.reshape(2, NC * NS, JPT, CH)
    zeros = jnp.zeros((N, D), jnp.float32)
    p0, p1 = _sc_segment_sum(x, ei, zeros)
    h2 = _tc_mlp(p0, p1, W1, b1.reshape(1, D), W2)
    q0, q1 = _sc_segment_sum(h2, ei, zeros)
    return _tc_bias(q0, q1, b2.reshape(1, D))
